# Initial kernel scaffold; baseline (speedup 1.0000x reference)
import jax
import jax.numpy as jnp
from jax.experimental import pallas as pl


def kernel(x, edge_index, W1, b1, W2, b2):
    raise NotImplementedError("write your pallas kernel here")


# speedup vs baseline: 19.7572x; 19.7572x over previous
import functools

import jax
import jax.numpy as jnp
from jax import lax
from jax.experimental import pallas as pl
from jax.experimental.pallas import tpu as pltpu
from jax.experimental.pallas import tpu_sc as plsc

N_NODES = 10000
N_EDGES = 320000
NC = 2
NS = 16
EDGES_PER_TILE = N_EDGES // (NC * NS)
R_LO = 624
R_HI = N_NODES - 15 * R_LO
CHUNK = 128
N_FULL = EDGES_PER_TILE // CHUNK
TAIL = EDGES_PER_TILE - N_FULL * CHUNK


def _deg_body(dst_hbm, zeros_hbm, ones_hbm, out_hbm, acc, idx_v, ones_v,
              idx_t, ones_t):
    cid = lax.axis_index("c")
    sid = lax.axis_index("s")

    @pl.when(sid < 15)
    def _():
        pltpu.sync_copy(zeros_hbm.at[pl.ds(0, R_LO)],
                        acc.at[pl.ds(sid * R_LO, R_LO)])

    @pl.when(sid == 15)
    def _():
        pltpu.sync_copy(zeros_hbm, acc.at[pl.ds(15 * R_LO, R_HI)])

    pltpu.sync_copy(ones_hbm, ones_v)
    pltpu.sync_copy(ones_hbm.at[pl.ds(0, TAIL)], ones_t)
    plsc.subcore_barrier()

    ebase = (cid * NS + sid) * EDGES_PER_TILE

    def step(j, carry):
        base = ebase + j * CHUNK
        pltpu.sync_copy(dst_hbm.at[pl.ds(base, CHUNK)], idx_v)
        pltpu.sync_copy(ones_v, acc.at[idx_v], add=True)
        return carry

    lax.fori_loop(0, N_FULL, step, 0)
    base = ebase + N_FULL * CHUNK
    pltpu.sync_copy(dst_hbm.at[pl.ds(base, TAIL)], idx_t)
    pltpu.sync_copy(ones_t, acc.at[idx_t], add=True)

    plsc.subcore_barrier()

    @pl.when(sid < 15)
    def _():
        pltpu.sync_copy(acc.at[pl.ds(sid * R_LO, R_LO)],
                        out_hbm.at[cid, pl.ds(sid * R_LO, R_LO)])

    @pl.when(sid == 15)
    def _():
        pltpu.sync_copy(acc.at[pl.ds(15 * R_LO, R_HI)],
                        out_hbm.at[cid, pl.ds(15 * R_LO, R_HI)])


def _make_deg_kernel():
    mesh = plsc.VectorSubcoreMesh(core_axis_name="c", subcore_axis_name="s",
                                  num_cores=NC, num_subcores=NS)
    return pl.kernel(
        _deg_body,
        out_type=jax.ShapeDtypeStruct((NC, N_NODES, 1), jnp.float32),
        mesh=mesh,
        scratch_types=[
            pltpu.VMEM_SHARED((N_NODES, 1), jnp.float32),
            pltpu.VMEM((CHUNK,), jnp.int32),
            pltpu.VMEM((CHUNK, 1), jnp.float32),
            pltpu.VMEM((TAIL,), jnp.int32),
            pltpu.VMEM((TAIL, 1), jnp.float32),
        ],
    )


def _agg_body(src_hbm, dst_hbm, tbl_hbm, zeros_hbm, out_hbm,
              acc, tbl_sh, idx_s, idx_d, rows, idx_st, idx_dt, rows_t, sem):
    cid = lax.axis_index("c")
    sid = lax.axis_index("s")

    @pl.when(sid < 15)
    def _():
        pltpu.sync_copy(zeros_hbm.at[pl.ds(0, R_LO)],
                        acc.at[pl.ds(sid * R_LO, R_LO)])
        pltpu.sync_copy(tbl_hbm.at[pl.ds(sid * R_LO, R_LO)],
                        tbl_sh.at[pl.ds(sid * R_LO, R_LO)])

    @pl.when(sid == 15)
    def _():
        pltpu.sync_copy(zeros_hbm, acc.at[pl.ds(15 * R_LO, R_HI)])
        pltpu.sync_copy(tbl_hbm.at[pl.ds(15 * R_LO, R_HI)],
                        tbl_sh.at[pl.ds(15 * R_LO, R_HI)])

    plsc.subcore_barrier()

    ebase = (cid * NS + sid) * EDGES_PER_TILE

    def step(j, carry):
        base = ebase + j * CHUNK
        pltpu.sync_copy(src_hbm.at[pl.ds(base, CHUNK)], idx_s)
        pltpu.sync_copy(dst_hbm.at[pl.ds(base, CHUNK)], idx_d)
        pltpu.async_copy(tbl_sh.at[idx_s], rows, sem).wait()
        pltpu.sync_copy(rows, acc.at[idx_d], add=True)
        return carry

    lax.fori_loop(0, N_FULL, step, 0)
    base = ebase + N_FULL * CHUNK
    pltpu.sync_copy(src_hbm.at[pl.ds(base, TAIL)], idx_st)
    pltpu.sync_copy(dst_hbm.at[pl.ds(base, TAIL)], idx_dt)
    pltpu.async_copy(tbl_sh.at[idx_st], rows_t, sem).wait()
    pltpu.sync_copy(rows_t, acc.at[idx_dt], add=True)

    plsc.subcore_barrier()

    @pl.when(sid < 15)
    def _():
        pltpu.sync_copy(acc.at[pl.ds(sid * R_LO, R_LO)],
                        out_hbm.at[cid, pl.ds(sid * R_LO, R_LO)])

    @pl.when(sid == 15)
    def _():
        pltpu.sync_copy(acc.at[pl.ds(15 * R_LO, R_HI)],
                        out_hbm.at[cid, pl.ds(15 * R_LO, R_HI)])


def _make_agg_kernel(d):
    mesh = plsc.VectorSubcoreMesh(core_axis_name="c", subcore_axis_name="s",
                                  num_cores=NC, num_subcores=NS)
    return pl.kernel(
        _agg_body,
        out_type=jax.ShapeDtypeStruct((NC, N_NODES, d), jnp.float32),
        mesh=mesh,
        scratch_types=[
            pltpu.VMEM_SHARED((N_NODES, d), jnp.float32),
            pltpu.VMEM_SHARED((N_NODES, d), jnp.float32),
            pltpu.VMEM((CHUNK,), jnp.int32),
            pltpu.VMEM((CHUNK,), jnp.int32),
            pltpu.VMEM((CHUNK, d), jnp.float32),
            pltpu.VMEM((TAIL,), jnp.int32),
            pltpu.VMEM((TAIL,), jnp.int32),
            pltpu.VMEM((TAIL, d), jnp.float32),
            pltpu.SemaphoreType.DMA,
        ],
    )


def _tc_a_body(degp_ref, x_ref, w1_ref, h1s_ref, dinv_ref):
    deg = degp_ref[:, 0:1] + degp_ref[:, 1:2] + 1.0
    dinv = lax.rsqrt(deg)
    h1 = jnp.dot(x_ref[...], w1_ref[...], preferred_element_type=jnp.float32)
    h1s_ref[...] = h1 * dinv
    dinv_ref[...] = dinv


def _tc_b_body(a0_ref, a1_ref, h1s_ref, dinv_ref, b1_ref, w2_ref, h2s_ref):
    agg = a0_ref[...] + a1_ref[...] + h1s_ref[...]
    out1 = agg * dinv_ref[...] + b1_ref[...]
    z = jnp.maximum(out1, 0.0)
    h2 = jnp.dot(z, w2_ref[...], preferred_element_type=jnp.float32)
    h2s_ref[...] = h2 * dinv_ref[...]


def _tc_c_body(a0_ref, a1_ref, h2s_ref, dinv_ref, b2_ref, out_ref):
    agg = a0_ref[...] + a1_ref[...] + h2s_ref[...]
    o = agg * dinv_ref[...] + b2_ref[...]
    col = lax.broadcasted_iota(jnp.int32, o.shape, 1)
    valid = col < 3
    om = jnp.where(valid, o, -jnp.inf)
    m = jnp.max(om, axis=1, keepdims=True)
    e = jnp.where(valid, jnp.exp(o - m), 0.0)
    s = jnp.sum(e, axis=1, keepdims=True)
    out_ref[...] = o - m - jnp.log(s)


def kernel(x, edge_index, W1, b1, W2, b2):
    src = edge_index[0].astype(jnp.int32)
    dst = edge_index[1].astype(jnp.int32)

    zeros1 = jnp.zeros((R_HI, 1), jnp.float32)
    ones_c = jnp.ones((CHUNK, 1), jnp.float32)

    degp = _make_deg_kernel()(dst, zeros1, ones_c)
    degp2 = jnp.transpose(degp[:, :, 0])

    h1s, dinv = pl.pallas_call(
        _tc_a_body,
        out_shape=(
            jax.ShapeDtypeStruct((N_NODES, 64), jnp.float32),
            jax.ShapeDtypeStruct((N_NODES, 1), jnp.float32),
        ),
    )(degp2, x, W1)

    zeros64 = jnp.zeros((R_HI, 64), jnp.float32)
    agg1 = _make_agg_kernel(64)(src, dst, h1s, zeros64)

    W2p = jnp.zeros((64, 16), jnp.float32).at[:, :3].set(W2)
    b2p = jnp.zeros((1, 16), jnp.float32).at[0, :3].set(b2)

    h2s = pl.pallas_call(
        _tc_b_body,
        out_shape=jax.ShapeDtypeStruct((N_NODES, 16), jnp.float32),
    )(agg1[0], agg1[1], h1s, dinv, b1.reshape(1, 64), W2p)

    zeros16 = jnp.zeros((R_HI, 16), jnp.float32)
    agg2 = _make_agg_kernel(16)(src, dst, h2s, zeros16)

    out16 = pl.pallas_call(
        _tc_c_body,
        out_shape=jax.ShapeDtypeStruct((N_NODES, 16), jnp.float32),
    )(agg2[0], agg2[1], h2s, dinv, b2p)

    return out16[:, :3]


# speedup vs baseline: 28.4485x; 1.4399x over previous
import jax
import jax.numpy as jnp
from jax import lax
from jax.experimental import pallas as pl
from jax.experimental.pallas import tpu as pltpu
from jax.experimental.pallas import tpu_sc as plsc

N_NODES = 10000
N_EDGES = 320000
NC = 2
NS = 16
EDGES_PER_TILE = N_EDGES // (NC * NS)
R_LO = 624
R_HI = N_NODES - 15 * R_LO
CHUNK = 128
N_FULL = EDGES_PER_TILE // CHUNK
TAIL = EDGES_PER_TILE - N_FULL * CHUNK


def _zero_rows(sid, zeros_hbm, sh_dst):
    @pl.when(sid < 15)
    def _():
        pltpu.sync_copy(zeros_hbm.at[pl.ds(0, R_LO)],
                        sh_dst.at[pl.ds(sid * R_LO, R_LO)])

    @pl.when(sid == 15)
    def _():
        pltpu.sync_copy(zeros_hbm, sh_dst.at[pl.ds(15 * R_LO, R_HI)])


def _stage_rows(sid, hbm_src, sh_dst):
    @pl.when(sid < 15)
    def _():
        pltpu.sync_copy(hbm_src.at[pl.ds(sid * R_LO, R_LO)],
                        sh_dst.at[pl.ds(sid * R_LO, R_LO)])

    @pl.when(sid == 15)
    def _():
        pltpu.sync_copy(hbm_src.at[pl.ds(15 * R_LO, R_HI)],
                        sh_dst.at[pl.ds(15 * R_LO, R_HI)])


def _writeout_rows(cid, sid, sh_src, hbm_dst):
    @pl.when(sid < 15)
    def _():
        pltpu.sync_copy(sh_src.at[pl.ds(sid * R_LO, R_LO)],
                        hbm_dst.at[cid, pl.ds(sid * R_LO, R_LO)])

    @pl.when(sid == 15)
    def _():
        pltpu.sync_copy(sh_src.at[pl.ds(15 * R_LO, R_HI)],
                        hbm_dst.at[cid, pl.ds(15 * R_LO, R_HI)])


def _deg_body(dst_hbm, zeros_hbm, ones_hbm, out_hbm, acc,
              didx0, didx1, dst_t, ones_v, ones_t,
              isem0, isem1, ssem0, ssem1):
    cid = lax.axis_index("c")
    sid = lax.axis_index("s")
    wid = cid * NS + sid
    ebase = wid * EDGES_PER_TILE

    _zero_rows(sid, zeros_hbm, acc)
    pltpu.sync_copy(ones_hbm, ones_v)
    pltpu.sync_copy(ones_hbm.at[pl.ds(0, TAIL)], ones_t)
    plsc.subcore_barrier()

    didx = (didx0, didx1)
    isem = (isem0, isem1)
    ssem = (ssem0, ssem1)

    pltpu.async_copy(dst_hbm.at[pl.ds(ebase, CHUNK)], didx0, isem0)

    def pair(k, carry):
        for b in (0, 1):
            j = 2 * k + b
            o = 1 - b

            def advance():
                pltpu.make_async_copy(ones_v, acc.at[didx[o]], ssem[o]).wait()
                pltpu.async_copy(
                    dst_hbm.at[pl.ds(ebase + (j + 1) * CHUNK, CHUNK)],
                    didx[o], isem[o])

            if b == 0:
                @pl.when(k >= 1)
                def _():
                    advance()

                @pl.when(k == 0)
                def _():
                    pltpu.async_copy(
                        dst_hbm.at[pl.ds(ebase + CHUNK, CHUNK)],
                        didx[o], isem[o])
            else:
                @pl.when(k < N_FULL // 2 - 1)
                def _():
                    advance()

            pltpu.make_async_copy(
                dst_hbm.at[pl.ds(ebase, CHUNK)], didx[b], isem[b]).wait()
            pltpu.async_copy(ones_v, acc.at[didx[b]], ssem[b], add=True)
        return carry

    lax.fori_loop(0, N_FULL // 2, pair, 0)
    pltpu.make_async_copy(ones_v, acc.at[didx0], ssem0).wait()
    pltpu.make_async_copy(ones_v, acc.at[didx1], ssem1).wait()

    pltpu.sync_copy(dst_hbm.at[pl.ds(ebase + N_FULL * CHUNK, TAIL)], dst_t)
    pltpu.sync_copy(ones_t, acc.at[dst_t], add=True)

    plsc.subcore_barrier()
    _writeout_rows(cid, sid, acc, out_hbm)


def _make_deg_kernel():
    mesh = plsc.VectorSubcoreMesh(core_axis_name="c", subcore_axis_name="s",
                                  num_cores=NC, num_subcores=NS)
    return pl.kernel(
        _deg_body,
        out_type=jax.ShapeDtypeStruct((NC, N_NODES, 1), jnp.float32),
        mesh=mesh,
        scratch_types=[
            pltpu.VMEM_SHARED((N_NODES, 1), jnp.float32),
            pltpu.VMEM((CHUNK,), jnp.int32),
            pltpu.VMEM((CHUNK,), jnp.int32),
            pltpu.VMEM((TAIL,), jnp.int32),
            pltpu.VMEM((CHUNK, 1), jnp.float32),
            pltpu.VMEM((TAIL, 1), jnp.float32),
            pltpu.SemaphoreType.DMA,
            pltpu.SemaphoreType.DMA,
            pltpu.SemaphoreType.DMA,
            pltpu.SemaphoreType.DMA,
        ],
    )


def _agg_body(src_hbm, dst_hbm, tbl_hbm, zeros_hbm, out_hbm,
              acc, tbl_sh, sidx0, sidx1, didx0, didx1, src_t, dst_t,
              rows0, rows1, rows_t,
              isem0, isem1, gsem0, gsem1, ssem0, ssem1):
    cid = lax.axis_index("c")
    sid = lax.axis_index("s")
    wid = cid * NS + sid
    ebase = wid * EDGES_PER_TILE

    _zero_rows(sid, zeros_hbm, acc)
    _stage_rows(sid, tbl_hbm, tbl_sh)
    plsc.subcore_barrier()

    sidx = (sidx0, sidx1)
    didx = (didx0, didx1)
    rows = (rows0, rows1)
    isem = (isem0, isem1)
    gsem = (gsem0, gsem1)
    ssem = (ssem0, ssem1)

    def load_idx(j, b):
        base = ebase + j * CHUNK
        pltpu.async_copy(src_hbm.at[pl.ds(base, CHUNK)], sidx[b], isem[b])
        pltpu.async_copy(dst_hbm.at[pl.ds(base, CHUNK)], didx[b], isem[b])

    def wait_idx(b):
        pltpu.make_async_copy(
            src_hbm.at[pl.ds(ebase, CHUNK)], sidx[b], isem[b]).wait()
        pltpu.make_async_copy(
            dst_hbm.at[pl.ds(ebase, CHUNK)], didx[b], isem[b]).wait()

    load_idx(0, 0)
    wait_idx(0)
    pltpu.async_copy(tbl_sh.at[sidx0], rows0, gsem0)

    def pair(k, carry):
        for b in (0, 1):
            j = 2 * k + b
            o = 1 - b

            def advance():
                pltpu.make_async_copy(
                    rows[o], acc.at[didx[o]], ssem[o]).wait()
                load_idx(j + 1, o)

            if b == 0:
                @pl.when(k >= 1)
                def _():
                    advance()

                @pl.when(k == 0)
                def _():
                    load_idx(1, o)
            else:
                @pl.when(k < N_FULL // 2 - 1)
                def _():
                    advance()

            pltpu.make_async_copy(tbl_sh.at[sidx[b]], rows[b],
                                  gsem[b]).wait()
            pltpu.async_copy(rows[b], acc.at[didx[b]], ssem[b], add=True)

            if b == 0:
                wait_idx(o)
                pltpu.async_copy(tbl_sh.at[sidx[o]], rows[o], gsem[o])
            else:
                @pl.when(k < N_FULL // 2 - 1)
                def _():
                    wait_idx(o)
                    pltpu.async_copy(tbl_sh.at[sidx[o]], rows[o], gsem[o])
        return carry

    lax.fori_loop(0, N_FULL // 2, pair, 0)
    pltpu.make_async_copy(rows0, acc.at[didx0], ssem0).wait()
    pltpu.make_async_copy(rows1, acc.at[didx1], ssem1).wait()

    base = ebase + N_FULL * CHUNK
    pltpu.sync_copy(src_hbm.at[pl.ds(base, TAIL)], src_t)
    pltpu.sync_copy(dst_hbm.at[pl.ds(base, TAIL)], dst_t)
    pltpu.async_copy(tbl_sh.at[src_t], rows_t, gsem0).wait()
    pltpu.sync_copy(rows_t, acc.at[dst_t], add=True)

    plsc.subcore_barrier()
    _writeout_rows(cid, sid, acc, out_hbm)


def _make_agg_kernel(d):
    mesh = plsc.VectorSubcoreMesh(core_axis_name="c", subcore_axis_name="s",
                                  num_cores=NC, num_subcores=NS)
    return pl.kernel(
        _agg_body,
        out_type=jax.ShapeDtypeStruct((NC, N_NODES, d), jnp.float32),
        mesh=mesh,
        scratch_types=[
            pltpu.VMEM_SHARED((N_NODES, d), jnp.float32),
            pltpu.VMEM_SHARED((N_NODES, d), jnp.float32),
            pltpu.VMEM((CHUNK,), jnp.int32),
            pltpu.VMEM((CHUNK,), jnp.int32),
            pltpu.VMEM((CHUNK,), jnp.int32),
            pltpu.VMEM((CHUNK,), jnp.int32),
            pltpu.VMEM((TAIL,), jnp.int32),
            pltpu.VMEM((TAIL,), jnp.int32),
            pltpu.VMEM((CHUNK, d), jnp.float32),
            pltpu.VMEM((CHUNK, d), jnp.float32),
            pltpu.VMEM((TAIL, d), jnp.float32),
            pltpu.SemaphoreType.DMA,
            pltpu.SemaphoreType.DMA,
            pltpu.SemaphoreType.DMA,
            pltpu.SemaphoreType.DMA,
            pltpu.SemaphoreType.DMA,
            pltpu.SemaphoreType.DMA,
        ],
    )


def _tc_a_body(degp_ref, x_ref, w1_ref, h1s_ref, dinv_ref):
    deg = degp_ref[:, 0:1] + degp_ref[:, 1:2] + 1.0
    dinv = lax.rsqrt(deg)
    h1 = jnp.dot(x_ref[...], w1_ref[...], preferred_element_type=jnp.float32)
    h1s_ref[...] = h1 * dinv
    dinv_ref[...] = dinv


def _tc_b_body(a0_ref, a1_ref, h1s_ref, dinv_ref, b1_ref, w2_ref, h2s_ref):
    agg = a0_ref[...] + a1_ref[...] + h1s_ref[...]
    out1 = agg * dinv_ref[...] + b1_ref[...]
    z = jnp.maximum(out1, 0.0)
    h2 = jnp.dot(z, w2_ref[...], preferred_element_type=jnp.float32)
    h2s_ref[...] = h2 * dinv_ref[...]


def _tc_c_body(a0_ref, a1_ref, h2s_ref, dinv_ref, b2_ref, out_ref):
    agg = a0_ref[...] + a1_ref[...] + h2s_ref[...]
    o = agg * dinv_ref[...] + b2_ref[...]
    col = lax.broadcasted_iota(jnp.int32, o.shape, 1)
    valid = col < 3
    om = jnp.where(valid, o, -jnp.inf)
    m = jnp.max(om, axis=1, keepdims=True)
    e = jnp.where(valid, jnp.exp(o - m), 0.0)
    s = jnp.sum(e, axis=1, keepdims=True)
    out_ref[...] = o - m - jnp.log(s)


def kernel(x, edge_index, W1, b1, W2, b2):
    src = edge_index[0].astype(jnp.int32)
    dst = edge_index[1].astype(jnp.int32)

    zeros1 = jnp.zeros((R_HI, 1), jnp.float32)
    ones_c = jnp.ones((CHUNK, 1), jnp.float32)

    degp = _make_deg_kernel()(dst, zeros1, ones_c)
    degp2 = jnp.transpose(degp[:, :, 0])

    h1s, dinv = pl.pallas_call(
        _tc_a_body,
        out_shape=(
            jax.ShapeDtypeStruct((N_NODES, 64), jnp.float32),
            jax.ShapeDtypeStruct((N_NODES, 1), jnp.float32),
        ),
    )(degp2, x, W1)

    zeros64 = jnp.zeros((R_HI, 64), jnp.float32)
    agg1 = _make_agg_kernel(64)(src, dst, h1s, zeros64)

    W2p = jnp.zeros((64, 16), jnp.float32).at[:, :3].set(W2)
    b2p = jnp.zeros((1, 16), jnp.float32).at[0, :3].set(b2)

    h2s = pl.pallas_call(
        _tc_b_body,
        out_shape=jax.ShapeDtypeStruct((N_NODES, 16), jnp.float32),
    )(agg1[0], agg1[1], h1s, dinv, b1.reshape(1, 64), W2p)

    zeros16 = jnp.zeros((R_HI, 16), jnp.float32)
    agg2 = _make_agg_kernel(16)(src, dst, h2s, zeros16)

    out16 = pl.pallas_call(
        _tc_c_body,
        out_shape=jax.ShapeDtypeStruct((N_NODES, 16), jnp.float32),
    )(agg2[0], agg2[1], h2s, dinv, b2p)

    return out16[:, :3]


# speedup vs baseline: 29.8681x; 1.0499x over previous
import jax
import jax.numpy as jnp
from jax import lax
from jax.experimental import pallas as pl
from jax.experimental.pallas import tpu as pltpu
from jax.experimental.pallas import tpu_sc as plsc

N_NODES = 10000
N_EDGES = 320000
NC = 2
NS = 16
EDGES_PER_TILE = N_EDGES // (NC * NS)
R_LO = 624
R_HI = N_NODES - 15 * R_LO
CHUNK = 128
N_FULL = EDGES_PER_TILE // CHUNK
TAIL = EDGES_PER_TILE - N_FULL * CHUNK
N_CH = 81
EPT_PAD = N_CH * CHUNK
TRASH = N_NODES
ACC_ROWS = N_NODES + 8
ZERO_HI = R_HI + 8


def _zero_rows(sid, zeros_hbm, sh_dst, hi):
    @pl.when(sid < 15)
    def _():
        pltpu.sync_copy(zeros_hbm.at[pl.ds(0, R_LO)],
                        sh_dst.at[pl.ds(sid * R_LO, R_LO)])

    @pl.when(sid == 15)
    def _():
        pltpu.sync_copy(zeros_hbm, sh_dst.at[pl.ds(15 * R_LO, hi)])


def _stage_rows(sid, hbm_src, sh_dst):
    @pl.when(sid < 15)
    def _():
        pltpu.sync_copy(hbm_src.at[pl.ds(sid * R_LO, R_LO)],
                        sh_dst.at[pl.ds(sid * R_LO, R_LO)])

    @pl.when(sid == 15)
    def _():
        pltpu.sync_copy(hbm_src.at[pl.ds(15 * R_LO, R_HI)],
                        sh_dst.at[pl.ds(15 * R_LO, R_HI)])


def _writeout_rows(cid, sid, sh_src, hbm_dst):
    @pl.when(sid < 15)
    def _():
        pltpu.sync_copy(sh_src.at[pl.ds(sid * R_LO, R_LO)],
                        hbm_dst.at[cid, pl.ds(sid * R_LO, R_LO)])

    @pl.when(sid == 15)
    def _():
        pltpu.sync_copy(sh_src.at[pl.ds(15 * R_LO, R_HI)],
                        hbm_dst.at[cid, pl.ds(15 * R_LO, R_HI)])


def _deg_body(dst_hbm, zeros_hbm, ones_hbm, out_hbm, acc,
              didx0, didx1, dst_t, ones_v, ones_t,
              isem0, isem1, ssem0, ssem1):
    cid = lax.axis_index("c")
    sid = lax.axis_index("s")
    wid = cid * NS + sid
    ebase = wid * EDGES_PER_TILE

    _zero_rows(sid, zeros_hbm, acc, R_HI)
    pltpu.sync_copy(ones_hbm, ones_v)
    pltpu.sync_copy(ones_hbm.at[pl.ds(0, TAIL)], ones_t)
    plsc.subcore_barrier()

    didx = (didx0, didx1)
    isem = (isem0, isem1)
    ssem = (ssem0, ssem1)

    pltpu.async_copy(dst_hbm.at[pl.ds(ebase, CHUNK)], didx0, isem0)

    def pair(k, carry):
        for b in (0, 1):
            j = 2 * k + b
            o = 1 - b

            def advance():
                pltpu.make_async_copy(ones_v, acc.at[didx[o]], ssem[o]).wait()
                pltpu.async_copy(
                    dst_hbm.at[pl.ds(ebase + (j + 1) * CHUNK, CHUNK)],
                    didx[o], isem[o])

            if b == 0:
                @pl.when(k >= 1)
                def _():
                    advance()

                @pl.when(k == 0)
                def _():
                    pltpu.async_copy(
                        dst_hbm.at[pl.ds(ebase + CHUNK, CHUNK)],
                        didx[o], isem[o])
            else:
                @pl.when(k < N_FULL // 2 - 1)
                def _():
                    advance()

            pltpu.make_async_copy(
                dst_hbm.at[pl.ds(ebase, CHUNK)], didx[b], isem[b]).wait()
            pltpu.async_copy(ones_v, acc.at[didx[b]], ssem[b], add=True)
        return carry

    lax.fori_loop(0, N_FULL // 2, pair, 0)
    pltpu.make_async_copy(ones_v, acc.at[didx0], ssem0).wait()
    pltpu.make_async_copy(ones_v, acc.at[didx1], ssem1).wait()

    pltpu.sync_copy(dst_hbm.at[pl.ds(ebase + N_FULL * CHUNK, TAIL)], dst_t)
    pltpu.sync_copy(ones_t, acc.at[dst_t], add=True)

    plsc.subcore_barrier()
    _writeout_rows(cid, sid, acc, out_hbm)


def _make_deg_kernel():
    mesh = plsc.VectorSubcoreMesh(core_axis_name="c", subcore_axis_name="s",
                                  num_cores=NC, num_subcores=NS)
    return pl.kernel(
        _deg_body,
        out_type=jax.ShapeDtypeStruct((NC, N_NODES, 1), jnp.float32),
        mesh=mesh,
        scratch_types=[
            pltpu.VMEM_SHARED((N_NODES, 1), jnp.float32),
            pltpu.VMEM((CHUNK,), jnp.int32),
            pltpu.VMEM((CHUNK,), jnp.int32),
            pltpu.VMEM((TAIL,), jnp.int32),
            pltpu.VMEM((CHUNK, 1), jnp.float32),
            pltpu.VMEM((TAIL, 1), jnp.float32),
            pltpu.SemaphoreType.DMA,
            pltpu.SemaphoreType.DMA,
            pltpu.SemaphoreType.DMA,
            pltpu.SemaphoreType.DMA,
        ],
    )


def _agg_body(src_hbm, dst_hbm, tbl_hbm, zeros_hbm, out_hbm,
              acc, tbl_sh, sidx0, sidx1, sidx2, didx0, didx1, didx2,
              rows0, rows1, rows2,
              isem0, isem1, isem2, gsem0, gsem1, gsem2,
              ssem0, ssem1, ssem2):
    cid = lax.axis_index("c")
    sid = lax.axis_index("s")
    wid = cid * NS + sid
    ebase = wid * EPT_PAD

    _zero_rows(sid, zeros_hbm, acc, ZERO_HI)
    _stage_rows(sid, tbl_hbm, tbl_sh)
    plsc.subcore_barrier()

    sidx = (sidx0, sidx1, sidx2)
    didx = (didx0, didx1, didx2)
    rows = (rows0, rows1, rows2)
    isem = (isem0, isem1, isem2)
    gsem = (gsem0, gsem1, gsem2)
    ssem = (ssem0, ssem1, ssem2)

    def load_idx(j, s):
        base = ebase + j * CHUNK
        pltpu.async_copy(src_hbm.at[pl.ds(base, CHUNK)], sidx[s], isem[s])
        pltpu.async_copy(dst_hbm.at[pl.ds(base, CHUNK)], didx[s], isem[s])

    def wait_idx(s):
        pltpu.make_async_copy(
            src_hbm.at[pl.ds(ebase, CHUNK)], sidx[s], isem[s]).wait()
        pltpu.make_async_copy(
            dst_hbm.at[pl.ds(ebase, CHUNK)], didx[s], isem[s]).wait()

    load_idx(0, 0)
    load_idx(1, 1)
    wait_idx(0)
    pltpu.async_copy(tbl_sh.at[sidx0], rows0, gsem0)

    def triple(k, carry):
        for b in (0, 1, 2):
            j = 3 * k + b
            r = b
            s1 = (b + 1) % 3
            s2 = (b + 2) % 3

            def gather_next():
                wait_idx(s1)
                pltpu.async_copy(tbl_sh.at[sidx[s1]], rows[s1], gsem[s1])

            def retire_and_prefetch():
                pltpu.make_async_copy(
                    rows[s2], acc.at[didx[s2]], ssem[s2]).wait()
                load_idx(j + 2, s2)

            if b == 0:
                @pl.when(k >= 1)
                def _():
                    retire_and_prefetch()

                @pl.when(k == 0)
                def _():
                    load_idx(2, s2)
            else:
                @pl.when(k < N_CH // 3 - 1)
                def _():
                    retire_and_prefetch()

                @pl.when(k == N_CH // 3 - 1)
                def _():
                    pltpu.make_async_copy(
                        rows[s2], acc.at[didx[s2]], ssem[s2]).wait()

            pltpu.make_async_copy(tbl_sh.at[sidx[r]], rows[r],
                                  gsem[r]).wait()
            if b == 2:
                @pl.when(k < N_CH // 3 - 1)
                def _():
                    gather_next()
            else:
                gather_next()
            pltpu.async_copy(rows[r], acc.at[didx[r]], ssem[r], add=True)
        return carry

    lax.fori_loop(0, N_CH // 3, triple, 0)
    pltpu.make_async_copy(rows2, acc.at[didx2], ssem2).wait()

    plsc.subcore_barrier()
    _writeout_rows(cid, sid, acc, out_hbm)


def _make_agg_kernel(d):
    mesh = plsc.VectorSubcoreMesh(core_axis_name="c", subcore_axis_name="s",
                                  num_cores=NC, num_subcores=NS)
    return pl.kernel(
        _agg_body,
        out_type=jax.ShapeDtypeStruct((NC, N_NODES, d), jnp.float32),
        mesh=mesh,
        scratch_types=[
            pltpu.VMEM_SHARED((ACC_ROWS, d), jnp.float32),
            pltpu.VMEM_SHARED((N_NODES, d), jnp.float32),
            pltpu.VMEM((CHUNK,), jnp.int32),
            pltpu.VMEM((CHUNK,), jnp.int32),
            pltpu.VMEM((CHUNK,), jnp.int32),
            pltpu.VMEM((CHUNK,), jnp.int32),
            pltpu.VMEM((CHUNK,), jnp.int32),
            pltpu.VMEM((CHUNK,), jnp.int32),
            pltpu.VMEM((CHUNK, d), jnp.float32),
            pltpu.VMEM((CHUNK, d), jnp.float32),
            pltpu.VMEM((CHUNK, d), jnp.float32),
            pltpu.SemaphoreType.DMA,
            pltpu.SemaphoreType.DMA,
            pltpu.SemaphoreType.DMA,
            pltpu.SemaphoreType.DMA,
            pltpu.SemaphoreType.DMA,
            pltpu.SemaphoreType.DMA,
            pltpu.SemaphoreType.DMA,
            pltpu.SemaphoreType.DMA,
            pltpu.SemaphoreType.DMA,
        ],
    )


def _tc_a_body(degp_ref, x_ref, w1_ref, h1s_ref, dinv_ref):
    deg = degp_ref[:, 0:1] + degp_ref[:, 1:2] + 1.0
    dinv = lax.rsqrt(deg)
    h1 = jnp.dot(x_ref[...], w1_ref[...], preferred_element_type=jnp.float32)
    h1s_ref[...] = h1 * dinv
    dinv_ref[...] = dinv


def _tc_b_body(a0_ref, a1_ref, h1s_ref, dinv_ref, b1_ref, w2_ref, h2s_ref):
    agg = a0_ref[...] + a1_ref[...] + h1s_ref[...]
    out1 = agg * dinv_ref[...] + b1_ref[...]
    z = jnp.maximum(out1, 0.0)
    h2 = jnp.dot(z, w2_ref[...], preferred_element_type=jnp.float32)
    h2s_ref[...] = h2 * dinv_ref[...]


def _tc_c_body(a0_ref, a1_ref, h2s_ref, dinv_ref, b2_ref, out_ref):
    agg = a0_ref[...] + a1_ref[...] + h2s_ref[...]
    o = agg * dinv_ref[...] + b2_ref[...]
    col = lax.broadcasted_iota(jnp.int32, o.shape, 1)
    valid = col < 3
    om = jnp.where(valid, o, -jnp.inf)
    m = jnp.max(om, axis=1, keepdims=True)
    e = jnp.where(valid, jnp.exp(o - m), 0.0)
    s = jnp.sum(e, axis=1, keepdims=True)
    out_ref[...] = o - m - jnp.log(s)


def kernel(x, edge_index, W1, b1, W2, b2):
    src = edge_index[0].astype(jnp.int32)
    dst = edge_index[1].astype(jnp.int32)
    pad_w = ((0, 0), (0, EPT_PAD - EDGES_PER_TILE))
    src_p = jnp.pad(src.reshape(NC * NS, EDGES_PER_TILE), pad_w).reshape(-1)
    dst_p = jnp.pad(dst.reshape(NC * NS, EDGES_PER_TILE), pad_w,
                    constant_values=TRASH).reshape(-1)

    zeros1 = jnp.zeros((R_HI, 1), jnp.float32)
    ones_c = jnp.ones((CHUNK, 1), jnp.float32)

    degp = _make_deg_kernel()(dst, zeros1, ones_c)
    degp2 = jnp.transpose(degp[:, :, 0])

    h1s, dinv = pl.pallas_call(
        _tc_a_body,
        out_shape=(
            jax.ShapeDtypeStruct((N_NODES, 64), jnp.float32),
            jax.ShapeDtypeStruct((N_NODES, 1), jnp.float32),
        ),
    )(degp2, x, W1)

    zeros64 = jnp.zeros((ZERO_HI, 64), jnp.float32)
    agg1 = _make_agg_kernel(64)(src_p, dst_p, h1s, zeros64)

    W2p = jnp.zeros((64, 16), jnp.float32).at[:, :3].set(W2)
    b2p = jnp.zeros((1, 16), jnp.float32).at[0, :3].set(b2)

    h2s = pl.pallas_call(
        _tc_b_body,
        out_shape=jax.ShapeDtypeStruct((N_NODES, 16), jnp.float32),
    )(agg1[0], agg1[1], h1s, dinv, b1.reshape(1, 64), W2p)

    zeros16 = jnp.zeros((ZERO_HI, 16), jnp.float32)
    agg2 = _make_agg_kernel(16)(src_p, dst_p, h2s, zeros16)

    out16 = pl.pallas_call(
        _tc_c_body,
        out_shape=jax.ShapeDtypeStruct((N_NODES, 16), jnp.float32),
    )(agg2[0], agg2[1], h2s, dinv, b2p)

    return out16[:, :3]
